# baseline (speedup 1.0000x reference)
import functools

import jax
import jax.numpy as jnp
import numpy as np
from jax import lax
from jax.experimental import pallas as pl
from jax.experimental.pallas import tpu as pltpu
from jax.experimental.pallas import tpu_sc as plsc

_N_COARSE = 10000
_N_FINE = 50000
_VAL_DIM = 128
_FE = 9
_NF = 64
_HALF = _N_COARSE // 2

_NC = 2
_NS = 16
_NW = _NC * _NS
_BPW = 1568
_C = 56
_NCHUNK = _BPW // _C
_PR = _C // 2


_HPAD = 5008


def _mm_body(x_ref, w_ref, o_ref):
    wk = w_ref[0]
    a = jnp.dot(x_ref[:_HALF], wk, preferred_element_type=jnp.float32)
    b = jnp.dot(x_ref[_HALF:], wk, preferred_element_type=jnp.float32)
    ab = jnp.concatenate([a, b], axis=1).astype(jnp.bfloat16)
    o_ref[0] = jnp.concatenate(
        [ab, jnp.zeros((_HPAD - _HALF, 2 * _NF), jnp.bfloat16)], axis=0)


def _project_table(table, w3):
    return pl.pallas_call(
        _mm_body,
        grid=(_FE,),
        in_specs=[
            pl.BlockSpec((_N_COARSE, _VAL_DIM), lambda k: (0, 0)),
            pl.BlockSpec((1, _VAL_DIM, _NF), lambda k: (k, 0, 0)),
        ],
        out_specs=pl.BlockSpec((1, _HPAD, 2 * _NF), lambda k: (k, 0, 0)),
        out_shape=jax.ShapeDtypeStruct((_FE, _HPAD, 2 * _NF), jnp.bfloat16),
    )(table, w3)


def _sc_gather_sum(p_flat, idx_r):
    mesh = plsc.VectorSubcoreMesh(core_axis_name="c", subcore_axis_name="s")

    @functools.partial(
        pl.kernel,
        mesh=mesh,
        out_type=jax.ShapeDtypeStruct((_N_FINE // 2, 2 * _NF), jnp.float32),
        scratch_types=[
            pltpu.VMEM((_FE, _BPW), jnp.int32),
            pltpu.VMEM((_FE * _C, _NF), jnp.bfloat16),
            pltpu.VMEM((_FE * _C, _NF), jnp.bfloat16),
            pltpu.VMEM((_PR, 2 * _NF), jnp.float32),
            pltpu.VMEM((_PR, 2 * _NF), jnp.float32),
            pltpu.SemaphoreType.DMA,
            pltpu.SemaphoreType.DMA,
            pltpu.SemaphoreType.DMA,
            pltpu.SemaphoreType.DMA,
            pltpu.SemaphoreType.DMA,
        ],
        compiler_params=pltpu.CompilerParams(
            use_tc_tiling_on_sc=False, needs_layout_passes=False),
    )
    def sc_fn(p_hbm, idx_hbm, out_hbm, idxb,
              rows0, rows1, outb0, outb1, si, sg0, sg1, so0, so1):
        w = lax.axis_index("s") * _NC + lax.axis_index("c")
        start = jnp.where(w < _NW - 1, w * _BPW, _N_FINE - _BPW)

        for k in range(_FE):
            pltpu.async_copy(idx_hbm.at[pl.ds(k * _N_FINE + start, _BPW)],
                             idxb.at[k], si)
        pltpu.make_async_copy(idx_hbm.at[pl.ds(0, _FE * _BPW)], idxb, si).wait()

        def fire(c, rows, sg):
            for k in range(_FE):
                pltpu.async_copy(p_hbm.at[idxb.at[k, pl.ds(c * _C, _C)]],
                                 rows.at[pl.ds(k * _C, _C)], sg)

        def wait_rows(rows, sg):
            pltpu.make_async_copy(p_hbm.at[pl.ds(0, _FE * _C)], rows, sg).wait()

        def wait_store(outb, so):
            pltpu.make_async_copy(outb, out_hbm.at[pl.ds(0, _PR)], so).wait()

        def accum(rows, outb):
            @plsc.parallel_loop(0, _PR)
            def _(u):
                for h in range(2):
                    i = 2 * u + h
                    for j in range(_NF // 32):
                        sj = pl.ds(j * 32, 32)
                        x = rows[i, sj]
                        acc_a, acc_b = plsc.unpack(
                            x, format=plsc.PackFormat.INTERLEAVED)
                        for k in range(1, _FE):
                            xa, xb = plsc.unpack(
                                rows[k * _C + i, sj],
                                format=plsc.PackFormat.INTERLEAVED)
                            acc_a = acc_a + xa
                            acc_b = acc_b + xb
                        outb[u, pl.ds(h * _NF + j * 32, 16)] = acc_a
                        outb[u, pl.ds(h * _NF + j * 32 + 16, 16)] = acc_b

        bufs = ((rows0, outb0, sg0, so0), (rows1, outb1, sg1, so1))
        fire(0, rows0, sg0)
        fire(1, rows1, sg1)

        def step(t2, carry):
            for b, (rows, outb, sg, so) in enumerate(bufs):
                c = 2 * t2 + b
                wait_rows(rows, sg)

                @pl.when(t2 > 0)
                def _():
                    wait_store(outb, so)

                accum(rows, outb)
                pltpu.async_copy(
                    outb,
                    out_hbm.at[pl.ds(start // 2 + c * _PR, _PR)], so)

                @pl.when(t2 < _NCHUNK // 2 - 1)
                def _():
                    fire(c + 2, rows, sg)
            return carry

        lax.fori_loop(0, _NCHUNK // 2, step, 0)
        wait_store(outb0, so0)
        wait_store(outb1, so1)

    return sc_fn(p_flat, idx_r)


def _feature_perm():
    p = np.empty(_NF, np.int32)
    for j in range(_NF // 32):
        for m in range(16):
            p[32 * j + 2 * m] = 32 * j + m
            p[32 * j + 2 * m + 1] = 32 * j + 16 + m
    return p


def kernel(lattice_coarse_values, neighbor_indices, weight):
    table = lattice_coarse_values
    w3 = weight.reshape(_FE, _VAL_DIM, _NF)[:, :, _feature_perm()]
    p3 = _project_table(table, w3)
    p_flat = p3.reshape(_FE * _HPAD * 2, _NF)

    v = neighbor_indices.T.astype(jnp.int32)
    v2 = v * 2
    idx2 = jnp.where(v < _HALF, v2, v2 - (_N_COARSE - 1)) + \
        (jnp.arange(_FE, dtype=jnp.int32) * (2 * _HPAD))[:, None]
    idx_r = idx2.reshape(_FE * _N_FINE)

    out = _sc_gather_sum(p_flat, idx_r)
    return out.reshape(_N_FINE, _NF)


# speedup vs baseline: 1.2521x; 1.2521x over previous
import functools

import jax
import jax.numpy as jnp
from jax import lax
from jax.experimental import pallas as pl
from jax.experimental.pallas import tpu as pltpu
from jax.experimental.pallas import tpu_sc as plsc

_N_COARSE = 10000
_N_FINE = 50000
_VAL_DIM = 128
_FE = 9
_NF = 64
_HALF = _N_COARSE // 2

_NC = 2
_NS = 16
_NW = _NC * _NS
_BPW = 1568
_C = 56
_NCHUNK = _BPW // _C
_PR = _C // 2


def _mm_body(x_ref, w_ref, o_ref):
    wk = w_ref[0].astype(jnp.bfloat16)
    x = x_ref[...].astype(jnp.bfloat16)
    a = jnp.dot(x[:_HALF], wk, preferred_element_type=jnp.float32)
    b = jnp.dot(x[_HALF:], wk, preferred_element_type=jnp.float32)
    o_ref[...] = jnp.concatenate([a, b], axis=1)


def _project_table(table, w3):
    return pl.pallas_call(
        _mm_body,
        grid=(_FE,),
        in_specs=[
            pl.BlockSpec((_N_COARSE, _VAL_DIM), lambda k: (0, 0)),
            pl.BlockSpec((1, _VAL_DIM, _NF), lambda k: (k, 0, 0)),
        ],
        out_specs=pl.BlockSpec((_HALF, 2 * _NF), lambda k: (k, 0)),
        out_shape=jax.ShapeDtypeStruct((_FE * _HALF, 2 * _NF), jnp.float32),
    )(table, w3)


def _sc_gather_sum(p_flat, idx_r):
    mesh = plsc.VectorSubcoreMesh(core_axis_name="c", subcore_axis_name="s")

    @functools.partial(
        pl.kernel,
        mesh=mesh,
        out_type=jax.ShapeDtypeStruct((_N_FINE, _NF), jnp.float32),
        scratch_types=[
            pltpu.VMEM((_FE, _BPW), jnp.int32),
            pltpu.VMEM((_FE * _C, _NF), jnp.float32),
            pltpu.VMEM((_FE * _C, _NF), jnp.float32),
            pltpu.VMEM((_C, _NF), jnp.float32),
            pltpu.VMEM((_C, _NF), jnp.float32),
            pltpu.SemaphoreType.DMA,
            pltpu.SemaphoreType.DMA,
            pltpu.SemaphoreType.DMA,
            pltpu.SemaphoreType.DMA,
            pltpu.SemaphoreType.DMA,
        ],
        compiler_params=pltpu.CompilerParams(use_tc_tiling_on_sc=False),
    )
    def sc_fn(p_hbm, idx_hbm, out_hbm, idxb,
              rows0, rows1, outb0, outb1, si, sg0, sg1, so0, so1):
        w = lax.axis_index("s") * _NC + lax.axis_index("c")
        start = jnp.where(w < _NW - 1, w * _BPW, _N_FINE - _BPW)

        for k in range(_FE):
            pltpu.async_copy(idx_hbm.at[pl.ds(k * _N_FINE + start, _BPW)],
                             idxb.at[k], si)
        pltpu.make_async_copy(idx_hbm.at[pl.ds(0, _FE * _BPW)], idxb, si).wait()

        def fire(c, rows, sg):
            for k in range(_FE):
                pltpu.async_copy(p_hbm.at[idxb.at[k, pl.ds(c * _C, _C)]],
                                 rows.at[pl.ds(k * _C, _C)], sg)

        def wait_rows(rows, sg):
            pltpu.make_async_copy(p_hbm.at[pl.ds(0, _FE * _C)], rows, sg).wait()

        def wait_store(outb, so):
            pltpu.make_async_copy(outb, out_hbm.at[pl.ds(0, _C)], so).wait()

        def accum(rows, outb):
            @plsc.parallel_loop(0, _C)
            def _(i):
                for j in range(_NF // 16):
                    sj = pl.ds(j * 16, 16)
                    acc = rows[i, sj]
                    for k in range(1, _FE):
                        acc = acc + rows[k * _C + i, sj]
                    outb[i, sj] = acc

        bufs = ((rows0, outb0, sg0, so0), (rows1, outb1, sg1, so1))
        fire(0, rows0, sg0)
        fire(1, rows1, sg1)

        def step(t2, carry):
            for b, (rows, outb, sg, so) in enumerate(bufs):
                c = 2 * t2 + b
                wait_rows(rows, sg)

                @pl.when(t2 > 0)
                def _():
                    wait_store(outb, so)

                accum(rows, outb)
                pltpu.async_copy(
                    outb,
                    out_hbm.at[pl.ds(start + c * _C, _C)], so)

                @pl.when(t2 < _NCHUNK // 2 - 1)
                def _():
                    fire(c + 2, rows, sg)
            return carry

        lax.fori_loop(0, _NCHUNK // 2, step, 0)
        wait_store(outb0, so0)
        wait_store(outb1, so1)

    return sc_fn(p_flat, idx_r)


def kernel(lattice_coarse_values, neighbor_indices, weight):
    table = lattice_coarse_values
    w3 = weight.reshape(_FE, _VAL_DIM, _NF)
    p2 = _project_table(table, w3)
    p_flat = p2.reshape(_N_COARSE * _FE, _NF)

    v = neighbor_indices.T.astype(jnp.int32)
    v2 = v * 2
    idx2 = jnp.where(v < _HALF, v2, v2 - (_N_COARSE - 1)) + \
        (jnp.arange(_FE, dtype=jnp.int32) * _N_COARSE)[:, None]
    idx_r = idx2.reshape(_FE * _N_FINE)

    return _sc_gather_sum(p_flat, idx_r)


# speedup vs baseline: 1.2794x; 1.0218x over previous
import functools

import jax
import jax.numpy as jnp
from jax import lax
from jax.experimental import pallas as pl
from jax.experimental.pallas import tpu as pltpu
from jax.experimental.pallas import tpu_sc as plsc

_N_COARSE = 10000
_N_FINE = 50000
_VAL_DIM = 128
_FE = 9
_NF = 64
_HALF = _N_COARSE // 2

_NC = 2
_NS = 16
_NW = _NC * _NS
_BPW = 1568
_C = 56
_NCHUNK = _BPW // _C
_PR = _C // 2


def _mm_body(x_ref, w_ref, o_ref):
    wk = w_ref[0].astype(jnp.bfloat16)
    x = x_ref[...].astype(jnp.bfloat16)
    a = jnp.dot(x[:_HALF], wk, preferred_element_type=jnp.float32)
    b = jnp.dot(x[_HALF:], wk, preferred_element_type=jnp.float32)
    o_ref[...] = jnp.concatenate([a, b], axis=1)


def _project_table(table, w3):
    return pl.pallas_call(
        _mm_body,
        grid=(_FE,),
        in_specs=[
            pl.BlockSpec((_N_COARSE, _VAL_DIM), lambda k: (0, 0)),
            pl.BlockSpec((1, _VAL_DIM, _NF), lambda k: (k, 0, 0)),
        ],
        out_specs=pl.BlockSpec((_HALF, 2 * _NF), lambda k: (k, 0)),
        out_shape=jax.ShapeDtypeStruct((_FE * _HALF, 2 * _NF), jnp.float32),
    )(table, w3)


def _sc_gather_sum(p_flat, idx_r):
    mesh = plsc.VectorSubcoreMesh(core_axis_name="c", subcore_axis_name="s")

    @functools.partial(
        pl.kernel,
        mesh=mesh,
        out_type=jax.ShapeDtypeStruct((_N_FINE, _NF), jnp.float32),
        scratch_types=[
            pltpu.VMEM((_FE, _BPW), jnp.int32),
            pltpu.VMEM((_FE * _C, _NF), jnp.float32),
            pltpu.VMEM((_FE * _C, _NF), jnp.float32),
            pltpu.VMEM((_FE * _C, _NF), jnp.float32),
            pltpu.VMEM((_C, _NF), jnp.float32),
            pltpu.VMEM((_C, _NF), jnp.float32),
            pltpu.VMEM((_C, _NF), jnp.float32),
            pltpu.SemaphoreType.DMA,
            pltpu.SemaphoreType.DMA,
            pltpu.SemaphoreType.DMA,
            pltpu.SemaphoreType.DMA,
            pltpu.SemaphoreType.DMA,
            pltpu.SemaphoreType.DMA,
            pltpu.SemaphoreType.DMA,
        ],
        compiler_params=pltpu.CompilerParams(use_tc_tiling_on_sc=False),
    )
    def sc_fn(p_hbm, idx_hbm, out_hbm, idxb,
              rows0, rows1, rows2, outb0, outb1, outb2,
              si, sg0, sg1, sg2, so0, so1, so2):
        w = lax.axis_index("s") * _NC + lax.axis_index("c")
        start = jnp.where(w < _NW - 1, w * _BPW, _N_FINE - _BPW)

        for k in range(_FE):
            pltpu.async_copy(idx_hbm.at[pl.ds(k * _N_FINE + start, _BPW)],
                             idxb.at[k], si)
        pltpu.make_async_copy(idx_hbm.at[pl.ds(0, _FE * _BPW)], idxb, si).wait()

        def fire(c, rows, sg):
            for k in range(_FE):
                pltpu.async_copy(p_hbm.at[idxb.at[k, pl.ds(c * _C, _C)]],
                                 rows.at[pl.ds(k * _C, _C)], sg)

        def wait_rows(rows, sg):
            pltpu.make_async_copy(p_hbm.at[pl.ds(0, _FE * _C)], rows, sg).wait()

        def wait_store(outb, so):
            pltpu.make_async_copy(outb, out_hbm.at[pl.ds(0, _C)], so).wait()

        def accum(rows, outb):
            @plsc.parallel_loop(0, _C)
            def _(i):
                for j in range(_NF // 16):
                    sj = pl.ds(j * 16, 16)
                    acc = rows[i, sj]
                    for k in range(1, _FE):
                        acc = acc + rows[k * _C + i, sj]
                    outb[i, sj] = acc

        bufs = ((rows0, outb0, sg0, so0), (rows1, outb1, sg1, so1),
                (rows2, outb2, sg2, so2))
        fire(0, rows0, sg0)
        fire(1, rows1, sg1)
        fire(2, rows2, sg2)

        def body(c, t, rows, outb, sg, so):
            wait_rows(rows, sg)

            @pl.when(t > 0)
            def _():
                wait_store(outb, so)

            accum(rows, outb)
            pltpu.async_copy(
                outb, out_hbm.at[pl.ds(start + c * _C, _C)], so)

        def step(t, carry):
            for b, (rows, outb, sg, so) in enumerate(bufs):
                c = 3 * t + b
                body(c, t, rows, outb, sg, so)
                if b == 0:
                    fire(c + 3, rows, sg)
                else:
                    @pl.when(t < _NCHUNK // 3 - 1)
                    def _():
                        fire(c + 3, rows, sg)
            return carry

        lax.fori_loop(0, _NCHUNK // 3, step, 0)
        body(_NCHUNK - 1, 1, rows0, outb0, sg0, so0)
        wait_store(outb0, so0)
        wait_store(outb1, so1)
        wait_store(outb2, so2)

    return sc_fn(p_flat, idx_r)


def kernel(lattice_coarse_values, neighbor_indices, weight):
    table = lattice_coarse_values
    w3 = weight.reshape(_FE, _VAL_DIM, _NF)
    p2 = _project_table(table, w3)
    p_flat = p2.reshape(_N_COARSE * _FE, _NF)

    v = neighbor_indices.T.astype(jnp.int32)
    v2 = v * 2
    idx2 = jnp.where(v < _HALF, v2, v2 - (_N_COARSE - 1)) + \
        (jnp.arange(_FE, dtype=jnp.int32) * _N_COARSE)[:, None]
    idx_r = idx2.reshape(_FE * _N_FINE)

    return _sc_gather_sum(p_flat, idx_r)
